# baseline (speedup 1.0000x reference)
import functools

import jax
import jax.numpy as jnp
from jax import lax
from jax.experimental import pallas as pl
from jax.experimental.pallas import tpu as pltpu
from jax.experimental.pallas import tpu_sc as plsc

_B = 16384
_D = 768

_info = plsc.get_sparse_core_info()
_NC = _info.num_cores
_NS = _info.num_subcores
_NW = _NC * _NS
_BPW = _B // _NW
_CHUNK = 64
_NCHUNK = _BPW // _CHUNK

_mesh = plsc.VectorSubcoreMesh(core_axis_name="c", subcore_axis_name="s")


@functools.partial(
    pl.kernel,
    mesh=_mesh,
    out_type=jax.ShapeDtypeStruct((_B, _D), jnp.float32),
    scratch_types=[
        pltpu.VMEM((_BPW,), jnp.int32),
        pltpu.VMEM((2, _CHUNK, _D), jnp.float32),
        pltpu.SemaphoreType.DMA,
        pltpu.SemaphoreType.DMA,
    ],
)
def _emb_lookup(labels_hbm, table_hbm, out_hbm, idx_v, rows_v, gsem, ssem):
    wid = lax.axis_index("s") * _NC + lax.axis_index("c")
    base = wid * _BPW
    pltpu.sync_copy(labels_hbm.at[pl.ds(base, _BPW)], idx_v)

    def gather(c, buf):
        return pltpu.make_async_copy(
            table_hbm.at[idx_v.at[pl.ds(c * _CHUNK, _CHUNK)]],
            rows_v.at[buf],
            gsem,
        )

    def scatter(c, buf):
        return pltpu.make_async_copy(
            rows_v.at[buf],
            out_hbm.at[pl.ds(base + c * _CHUNK, _CHUNK)],
            ssem,
        )

    gather(0, 0).start()
    for c in range(_NCHUNK):
        buf = c % 2
        gather(c, buf).wait()
        if c + 1 < _NCHUNK:
            if c >= 1:
                scatter(c - 1, buf ^ 1).wait()
            gather(c + 1, buf ^ 1).start()
        scatter(c, buf).start()
    scatter(_NCHUNK - 2, (_NCHUNK - 2) % 2).wait()
    scatter(_NCHUNK - 1, (_NCHUNK - 1) % 2).wait()


def kernel(labels, table):
    return _emb_lookup(labels.astype(jnp.int32), table)
